# baseline (speedup 1.0000x reference)
import functools

import jax
import jax.numpy as jnp
from jax import lax
from jax.experimental import pallas as pl
from jax.experimental.pallas import tpu as pltpu
from jax.experimental.pallas import tpu_sc as plsc

N_NODES = 10000
N_EDGES = 160000
F_IN = 128
F_OUT = 32

NCORES = 2
NSUB = 16
NWORK = NCORES * NSUB
CHUNK = 128
EDGES_PER_TILE = 5120
NCHUNK = EDGES_PER_TILE // CHUNK
E_PAD = EDGES_PER_TILE * NWORK
NPAD = 10240
DUMMY = 10016
ROWS_PER_TILE = NPAD // NSUB

def _zero_rows(ref, nrows, ncols):
    zv = jnp.zeros((16,), jnp.float32)

    def body(i, carry):
        for c0 in range(0, ncols, 16):
            ref[i, pl.ds(c0, 16)] = zv
        return carry

    lax.fori_loop(0, nrows, body, 0)


def _sc_degrees_body(edges_hbm, out_hbm, src_idx, dst_idx, ones_v, zbuf,
                     acc_s, acc_d):
    c = lax.axis_index("c")
    s = lax.axis_index("s")
    wid = s * NCORES + c
    ones16 = jnp.ones((16,), jnp.float32)

    def fill_ones(i, carry):
        ones_v[i, :] = ones16
        return carry

    lax.fori_loop(0, CHUNK, fill_ones, 0)
    _zero_rows(zbuf, ROWS_PER_TILE, 16)

    row0 = s * ROWS_PER_TILE
    pltpu.sync_copy(zbuf, acc_s.at[pl.ds(row0, ROWS_PER_TILE)])
    pltpu.sync_copy(zbuf, acc_d.at[pl.ds(row0, ROWS_PER_TILE)])
    plsc.subcore_barrier()

    pltpu.sync_copy(edges_hbm.at[0, wid], src_idx)
    pltpu.sync_copy(edges_hbm.at[1, wid], dst_idx)

    def chunk(j, carry):
        pltpu.sync_copy(ones_v, acc_s.at[src_idx.at[j]], add=True)
        pltpu.sync_copy(ones_v, acc_d.at[dst_idx.at[j]], add=True)
        return carry

    lax.fori_loop(0, NCHUNK, chunk, 0)
    plsc.subcore_barrier()

    pltpu.sync_copy(acc_s.at[pl.ds(row0, ROWS_PER_TILE)],
                    out_hbm.at[c, 0, pl.ds(row0, ROWS_PER_TILE)])
    pltpu.sync_copy(acc_d.at[pl.ds(row0, ROWS_PER_TILE)],
                    out_hbm.at[c, 1, pl.ds(row0, ROWS_PER_TILE)])


NSLOT = 8
NBLK = NCHUNK // NSLOT


def _sc_aggregate_body(h_hbm, edges_hbm, out_hbm, src_idx, dst_idx, rows_v,
                       acc, sems):
    c = lax.axis_index("c")
    s = lax.axis_index("s")
    wid = s * NCORES + c
    row0 = s * ROWS_PER_TILE

    _zero_rows(rows_v.at[0], CHUNK, F_OUT)
    for m in range(ROWS_PER_TILE // CHUNK):
        pltpu.sync_copy(rows_v.at[0], acc.at[pl.ds(row0 + m * CHUNK, CHUNK)])
    pltpu.sync_copy(edges_hbm.at[0, wid], src_idx)
    pltpu.sync_copy(edges_hbm.at[1, wid], dst_idx)
    plsc.subcore_barrier()

    for k in range(NSLOT):
        pltpu.async_copy(h_hbm.at[src_idx.at[k]], rows_v.at[k], sems.at[k])

    def blk(bb, carry):
        j0 = bb * NSLOT
        for k in range(NSLOT):
            j = j0 + k
            pltpu.make_async_copy(h_hbm.at[src_idx.at[j]], rows_v.at[k],
                                  sems.at[k]).wait()
            pltpu.sync_copy(rows_v.at[k], acc.at[dst_idx.at[j]], add=True)

            @pl.when(bb < NBLK - 1)
            def _refire():
                pltpu.async_copy(h_hbm.at[src_idx.at[j + NSLOT]], rows_v.at[k],
                                 sems.at[k])
        return carry

    lax.fori_loop(0, NBLK, blk, 0)
    plsc.subcore_barrier()

    pltpu.sync_copy(acc.at[pl.ds(row0, ROWS_PER_TILE)],
                    out_hbm.at[c, pl.ds(row0, ROWS_PER_TILE)])


@functools.cache
def _sc_kernels():
    mesh = plsc.VectorSubcoreMesh(core_axis_name="c", subcore_axis_name="s",
                                  num_cores=NCORES, num_subcores=NSUB)
    degrees = pl.kernel(
        _sc_degrees_body,
        out_type=jax.ShapeDtypeStruct((NCORES, 2, NPAD, 16), jnp.float32),
        mesh=mesh,
        compiler_params=pltpu.CompilerParams(use_tc_tiling_on_sc=False),
        scratch_types=[
            pltpu.VMEM((NCHUNK, CHUNK), jnp.int32),
            pltpu.VMEM((NCHUNK, CHUNK), jnp.int32),
            pltpu.VMEM((CHUNK, 16), jnp.float32),
            pltpu.VMEM((ROWS_PER_TILE, 16), jnp.float32),
            pltpu.VMEM_SHARED((NPAD, 16), jnp.float32),
            pltpu.VMEM_SHARED((NPAD, 16), jnp.float32),
        ],
    )
    aggregate = pl.kernel(
        _sc_aggregate_body,
        out_type=jax.ShapeDtypeStruct((NCORES, NPAD, F_OUT), jnp.float32),
        mesh=mesh,
        compiler_params=pltpu.CompilerParams(use_tc_tiling_on_sc=False),
        scratch_types=[
            pltpu.VMEM((NCHUNK, CHUNK), jnp.int32),
            pltpu.VMEM((NCHUNK, CHUNK), jnp.int32),
            pltpu.VMEM((NSLOT, CHUNK, F_OUT), jnp.float32),
            pltpu.VMEM_SHARED((NPAD, F_OUT), jnp.float32),
            pltpu.SemaphoreType.DMA((NSLOT,)),
        ],
    )
    return degrees, aggregate


def _enc_body(feat_ref, dego_ref, w1_ref, out_ref):
    norm = lax.rsqrt(jnp.maximum(dego_ref[...], 1.0))
    h = feat_ref[...] * norm[:, None]
    out_ref[...] = jnp.dot(h, w1_ref[...], preferred_element_type=jnp.float32)


def _tc_encode(feat, deg_out, w1):
    blk = 1024
    grid = NPAD // blk
    return pl.pallas_call(
        _enc_body,
        grid=(grid,),
        in_specs=[
            pl.BlockSpec((blk, F_IN), lambda i: (i, 0)),
            pl.BlockSpec((blk,), lambda i: (i,)),
            pl.BlockSpec((F_IN, F_OUT), lambda i: (0, 0)),
        ],
        out_specs=pl.BlockSpec((blk, F_OUT), lambda i: (i, 0)),
        out_shape=jax.ShapeDtypeStruct((NPAD, F_OUT), jnp.float32),
    )(feat, deg_out, w1)


def _mid_body(agg_ref, degi_ref, dego_ref, b1_ref, out_ref):
    agg = agg_ref[0] + agg_ref[1]
    ni = lax.rsqrt(jnp.maximum(degi_ref[...], 1.0))
    no = lax.rsqrt(jnp.maximum(dego_ref[...], 1.0))
    h1 = agg * ni[:, None] + b1_ref[...][None, :]
    out_ref[...] = h1 * no[:, None]


def _tc_mid(aggp, deg_in, deg_out, b1):
    blk = 1024
    grid = NPAD // blk
    return pl.pallas_call(
        _mid_body,
        grid=(grid,),
        in_specs=[
            pl.BlockSpec((NCORES, blk, F_OUT), lambda i: (0, i, 0)),
            pl.BlockSpec((blk,), lambda i: (i,)),
            pl.BlockSpec((blk,), lambda i: (i,)),
            pl.BlockSpec((F_OUT,), lambda i: (0,)),
        ],
        out_specs=pl.BlockSpec((blk, F_OUT), lambda i: (i, 0)),
        out_shape=jax.ShapeDtypeStruct((NPAD, F_OUT), jnp.float32),
    )(aggp, deg_in, deg_out, b1)


def _fin_body(agg_ref, degi_ref, eps_ref, w2_ref, b2_ref, w3_ref, b3_ref,
              mu_ref, std_ref, z_ref):
    agg = agg_ref[0] + agg_ref[1]
    ni = lax.rsqrt(jnp.maximum(degi_ref[...], 1.0))
    aggn = agg * ni[:, None]
    mu = jnp.dot(aggn, w2_ref[...], preferred_element_type=jnp.float32) \
        + b2_ref[...][None, :]
    logvar = jnp.dot(aggn, w3_ref[...], preferred_element_type=jnp.float32) \
        + b3_ref[...][None, :]
    std = jnp.exp(logvar)
    mu_ref[...] = mu
    std_ref[...] = std
    z_ref[...] = eps_ref[...] * std + mu


def _tc_final(aggp, deg_in, eps, w2, b2, w3, b3):
    blk = 1024
    grid = pl.cdiv(N_NODES, blk)
    out = jax.ShapeDtypeStruct((N_NODES, F_OUT), jnp.float32)
    return pl.pallas_call(
        _fin_body,
        grid=(grid,),
        in_specs=[
            pl.BlockSpec((NCORES, blk, F_OUT), lambda i: (0, i, 0)),
            pl.BlockSpec((blk,), lambda i: (i,)),
            pl.BlockSpec((blk, F_OUT), lambda i: (i, 0)),
            pl.BlockSpec((F_OUT, F_OUT), lambda i: (0, 0)),
            pl.BlockSpec((F_OUT,), lambda i: (0,)),
            pl.BlockSpec((F_OUT, F_OUT), lambda i: (0, 0)),
            pl.BlockSpec((F_OUT,), lambda i: (0,)),
        ],
        out_specs=[
            pl.BlockSpec((blk, F_OUT), lambda i: (i, 0)),
            pl.BlockSpec((blk, F_OUT), lambda i: (i, 0)),
            pl.BlockSpec((blk, F_OUT), lambda i: (i, 0)),
        ],
        out_shape=[out, out, out],
    )(aggp, deg_in, eps, w2, b2, w3, b3)


def _dec_body(zi_ref, zj_ref, out_ref):
    prod = lax.dot_general(zi_ref[...], zj_ref[...],
                           (((1,), (1,)), ((), ())),
                           preferred_element_type=jnp.float32)
    out_ref[...] = 0.5 * jnp.tanh(0.5 * prod) + 0.5


def _tc_decode(z):
    bi, bj = 2048, 2048
    return pl.pallas_call(
        _dec_body,
        grid=(pl.cdiv(N_NODES, bi), pl.cdiv(N_NODES, bj)),
        in_specs=[
            pl.BlockSpec((bi, F_OUT), lambda i, j: (i, 0)),
            pl.BlockSpec((bj, F_OUT), lambda i, j: (j, 0)),
        ],
        out_specs=pl.BlockSpec((bi, bj), lambda i, j: (i, j)),
        out_shape=jax.ShapeDtypeStruct((N_NODES, N_NODES), jnp.float32),
    )(z, z)


def kernel(feat, edge_index, W1, b1, W2, b2, W3, b3, eps):
    ei = jnp.full((2, E_PAD), DUMMY, jnp.int32)
    ei = ei.at[:, :N_EDGES].set(edge_index)
    ei = ei.reshape(2, NWORK, NCHUNK, CHUNK)

    sc_degrees, sc_aggregate = _sc_kernels()
    degp = sc_degrees(ei)
    deg_out = degp[0, 0, :, 0] + degp[1, 0, :, 0]
    deg_in = degp[0, 1, :, 0] + degp[1, 1, :, 0]

    h = _tc_encode(feat, deg_out, W1)

    agg1 = sc_aggregate(h, ei)
    g = _tc_mid(agg1, deg_in, deg_out, b1)

    agg2 = sc_aggregate(g, ei)
    mu, std, z = _tc_final(agg2, deg_in, eps, W2, b2, W3, b3)

    adj = _tc_decode(z)
    return adj, mu, std


# speedup vs baseline: 1.1685x; 1.1685x over previous
import functools

import jax
import jax.numpy as jnp
from jax import lax
from jax.experimental import pallas as pl
from jax.experimental.pallas import tpu as pltpu
from jax.experimental.pallas import tpu_sc as plsc

N_NODES = 10000
N_EDGES = 160000
F_IN = 128
F_OUT = 32

NCORES = 2
NSUB = 16
NWORK = NCORES * NSUB
CHUNK = 128
EDGES_PER_TILE = 5120
NCHUNK = EDGES_PER_TILE // CHUNK
E_PAD = EDGES_PER_TILE * NWORK
NPAD = 10240
DUMMY = 10016
ROWS_PER_TILE = NPAD // NSUB

def _zero_rows(ref, nrows, ncols):
    zv = jnp.zeros((16,), jnp.float32)

    def body(i, carry):
        for c0 in range(0, ncols, 16):
            ref[i, pl.ds(c0, 16)] = zv
        return carry

    lax.fori_loop(0, nrows, body, 0)


def _sc_degrees_body(edges_hbm, out_hbm, src_idx, dst_idx, ones_v, zbuf,
                     acc_s, acc_d):
    c = lax.axis_index("c")
    s = lax.axis_index("s")
    wid = s * NCORES + c
    ones16 = jnp.ones((16,), jnp.float32)

    def fill_ones(i, carry):
        ones_v[i, :] = ones16
        return carry

    lax.fori_loop(0, CHUNK, fill_ones, 0)
    _zero_rows(zbuf, ROWS_PER_TILE, 16)

    row0 = s * ROWS_PER_TILE
    pltpu.sync_copy(zbuf, acc_s.at[pl.ds(row0, ROWS_PER_TILE)])
    pltpu.sync_copy(zbuf, acc_d.at[pl.ds(row0, ROWS_PER_TILE)])
    plsc.subcore_barrier()

    pltpu.sync_copy(edges_hbm.at[0, wid], src_idx)
    pltpu.sync_copy(edges_hbm.at[1, wid], dst_idx)

    def chunk(j, carry):
        pltpu.sync_copy(ones_v, acc_s.at[src_idx.at[j]], add=True)
        pltpu.sync_copy(ones_v, acc_d.at[dst_idx.at[j]], add=True)
        return carry

    lax.fori_loop(0, NCHUNK, chunk, 0)
    plsc.subcore_barrier()

    pltpu.sync_copy(acc_s.at[pl.ds(row0, ROWS_PER_TILE)],
                    out_hbm.at[c, 0, pl.ds(row0, ROWS_PER_TILE)])
    pltpu.sync_copy(acc_d.at[pl.ds(row0, ROWS_PER_TILE)],
                    out_hbm.at[c, 1, pl.ds(row0, ROWS_PER_TILE)])


NSLOT = 8
NBLK = NCHUNK // NSLOT


def _sc_aggregate_body(h_hbm, edges_hbm, out_hbm, src_idx, dst_idx, rows_v,
                       h_s, acc, sems):
    c = lax.axis_index("c")
    s = lax.axis_index("s")
    wid = s * NCORES + c
    row0 = s * ROWS_PER_TILE

    _zero_rows(rows_v.at[0], CHUNK, F_OUT)
    for m in range(ROWS_PER_TILE // CHUNK):
        pltpu.sync_copy(rows_v.at[0], acc.at[pl.ds(row0 + m * CHUNK, CHUNK)])
    pltpu.sync_copy(h_hbm.at[pl.ds(row0, ROWS_PER_TILE)],
                    h_s.at[pl.ds(row0, ROWS_PER_TILE)])
    pltpu.sync_copy(edges_hbm.at[0, wid], src_idx)
    pltpu.sync_copy(edges_hbm.at[1, wid], dst_idx)
    plsc.subcore_barrier()

    for k in range(NSLOT):
        pltpu.async_copy(h_s.at[src_idx.at[k]], rows_v.at[k], sems.at[k])

    def blk(bb, carry):
        j0 = bb * NSLOT
        for k in range(NSLOT):
            j = j0 + k
            pltpu.make_async_copy(h_s.at[src_idx.at[j]], rows_v.at[k],
                                  sems.at[k]).wait()
            pltpu.sync_copy(rows_v.at[k], acc.at[dst_idx.at[j]], add=True)

            @pl.when(bb < NBLK - 1)
            def _refire():
                pltpu.async_copy(h_s.at[src_idx.at[j + NSLOT]], rows_v.at[k],
                                 sems.at[k])
        return carry

    lax.fori_loop(0, NBLK, blk, 0)
    plsc.subcore_barrier()

    pltpu.sync_copy(acc.at[pl.ds(row0, ROWS_PER_TILE)],
                    out_hbm.at[c, pl.ds(row0, ROWS_PER_TILE)])


@functools.cache
def _sc_kernels():
    mesh = plsc.VectorSubcoreMesh(core_axis_name="c", subcore_axis_name="s",
                                  num_cores=NCORES, num_subcores=NSUB)
    degrees = pl.kernel(
        _sc_degrees_body,
        out_type=jax.ShapeDtypeStruct((NCORES, 2, NPAD, 16), jnp.float32),
        mesh=mesh,
        compiler_params=pltpu.CompilerParams(use_tc_tiling_on_sc=False),
        scratch_types=[
            pltpu.VMEM((NCHUNK, CHUNK), jnp.int32),
            pltpu.VMEM((NCHUNK, CHUNK), jnp.int32),
            pltpu.VMEM((CHUNK, 16), jnp.float32),
            pltpu.VMEM((ROWS_PER_TILE, 16), jnp.float32),
            pltpu.VMEM_SHARED((NPAD, 16), jnp.float32),
            pltpu.VMEM_SHARED((NPAD, 16), jnp.float32),
        ],
    )
    aggregate = pl.kernel(
        _sc_aggregate_body,
        out_type=jax.ShapeDtypeStruct((NCORES, NPAD, F_OUT), jnp.float32),
        mesh=mesh,
        compiler_params=pltpu.CompilerParams(use_tc_tiling_on_sc=False),
        scratch_types=[
            pltpu.VMEM((NCHUNK, CHUNK), jnp.int32),
            pltpu.VMEM((NCHUNK, CHUNK), jnp.int32),
            pltpu.VMEM((NSLOT, CHUNK, F_OUT), jnp.float32),
            pltpu.VMEM_SHARED((NPAD, F_OUT), jnp.float32),
            pltpu.VMEM_SHARED((NPAD, F_OUT), jnp.float32),
            pltpu.SemaphoreType.DMA((NSLOT,)),
        ],
    )
    return degrees, aggregate


def _enc_body(feat_ref, dego_ref, w1_ref, out_ref):
    norm = lax.rsqrt(jnp.maximum(dego_ref[...], 1.0))
    h = feat_ref[...] * norm[:, None]
    out_ref[...] = jnp.dot(h, w1_ref[...], preferred_element_type=jnp.float32)


def _tc_encode(feat, deg_out, w1):
    blk = 1024
    grid = NPAD // blk
    return pl.pallas_call(
        _enc_body,
        grid=(grid,),
        in_specs=[
            pl.BlockSpec((blk, F_IN), lambda i: (i, 0)),
            pl.BlockSpec((blk,), lambda i: (i,)),
            pl.BlockSpec((F_IN, F_OUT), lambda i: (0, 0)),
        ],
        out_specs=pl.BlockSpec((blk, F_OUT), lambda i: (i, 0)),
        out_shape=jax.ShapeDtypeStruct((NPAD, F_OUT), jnp.float32),
    )(feat, deg_out, w1)


def _mid_body(agg_ref, degi_ref, dego_ref, b1_ref, out_ref):
    agg = agg_ref[0] + agg_ref[1]
    ni = lax.rsqrt(jnp.maximum(degi_ref[...], 1.0))
    no = lax.rsqrt(jnp.maximum(dego_ref[...], 1.0))
    h1 = agg * ni[:, None] + b1_ref[...][None, :]
    out_ref[...] = h1 * no[:, None]


def _tc_mid(aggp, deg_in, deg_out, b1):
    blk = 1024
    grid = NPAD // blk
    return pl.pallas_call(
        _mid_body,
        grid=(grid,),
        in_specs=[
            pl.BlockSpec((NCORES, blk, F_OUT), lambda i: (0, i, 0)),
            pl.BlockSpec((blk,), lambda i: (i,)),
            pl.BlockSpec((blk,), lambda i: (i,)),
            pl.BlockSpec((F_OUT,), lambda i: (0,)),
        ],
        out_specs=pl.BlockSpec((blk, F_OUT), lambda i: (i, 0)),
        out_shape=jax.ShapeDtypeStruct((NPAD, F_OUT), jnp.float32),
    )(aggp, deg_in, deg_out, b1)


def _fin_body(agg_ref, degi_ref, eps_ref, w2_ref, b2_ref, w3_ref, b3_ref,
              mu_ref, std_ref, z_ref):
    agg = agg_ref[0] + agg_ref[1]
    ni = lax.rsqrt(jnp.maximum(degi_ref[...], 1.0))
    aggn = agg * ni[:, None]
    mu = jnp.dot(aggn, w2_ref[...], preferred_element_type=jnp.float32) \
        + b2_ref[...][None, :]
    logvar = jnp.dot(aggn, w3_ref[...], preferred_element_type=jnp.float32) \
        + b3_ref[...][None, :]
    std = jnp.exp(logvar)
    mu_ref[...] = mu
    std_ref[...] = std
    z_ref[...] = eps_ref[...] * std + mu


def _tc_final(aggp, deg_in, eps, w2, b2, w3, b3):
    blk = 1024
    grid = pl.cdiv(N_NODES, blk)
    out = jax.ShapeDtypeStruct((N_NODES, F_OUT), jnp.float32)
    return pl.pallas_call(
        _fin_body,
        grid=(grid,),
        in_specs=[
            pl.BlockSpec((NCORES, blk, F_OUT), lambda i: (0, i, 0)),
            pl.BlockSpec((blk,), lambda i: (i,)),
            pl.BlockSpec((blk, F_OUT), lambda i: (i, 0)),
            pl.BlockSpec((F_OUT, F_OUT), lambda i: (0, 0)),
            pl.BlockSpec((F_OUT,), lambda i: (0,)),
            pl.BlockSpec((F_OUT, F_OUT), lambda i: (0, 0)),
            pl.BlockSpec((F_OUT,), lambda i: (0,)),
        ],
        out_specs=[
            pl.BlockSpec((blk, F_OUT), lambda i: (i, 0)),
            pl.BlockSpec((blk, F_OUT), lambda i: (i, 0)),
            pl.BlockSpec((blk, F_OUT), lambda i: (i, 0)),
        ],
        out_shape=[out, out, out],
    )(aggp, deg_in, eps, w2, b2, w3, b3)


def _dec_body(zi_ref, zj_ref, out_ref):
    prod = lax.dot_general(zi_ref[...], zj_ref[...],
                           (((1,), (1,)), ((), ())),
                           preferred_element_type=jnp.float32)
    out_ref[...] = 0.5 * jnp.tanh(0.5 * prod) + 0.5


def _tc_decode(z):
    bi, bj = 2048, 2048
    return pl.pallas_call(
        _dec_body,
        grid=(pl.cdiv(N_NODES, bi), pl.cdiv(N_NODES, bj)),
        in_specs=[
            pl.BlockSpec((bi, F_OUT), lambda i, j: (i, 0)),
            pl.BlockSpec((bj, F_OUT), lambda i, j: (j, 0)),
        ],
        out_specs=pl.BlockSpec((bi, bj), lambda i, j: (i, j)),
        out_shape=jax.ShapeDtypeStruct((N_NODES, N_NODES), jnp.float32),
    )(z, z)


def kernel(feat, edge_index, W1, b1, W2, b2, W3, b3, eps):
    ei = jnp.full((2, E_PAD), DUMMY, jnp.int32)
    ei = ei.at[:, :N_EDGES].set(edge_index)
    ei = ei.reshape(2, NWORK, NCHUNK, CHUNK)

    sc_degrees, sc_aggregate = _sc_kernels()
    degp = sc_degrees(ei)
    deg_out = degp[0, 0, :, 0] + degp[1, 0, :, 0]
    deg_in = degp[0, 1, :, 0] + degp[1, 1, :, 0]

    h = _tc_encode(feat, deg_out, W1)

    agg1 = sc_aggregate(h, ei)
    g = _tc_mid(agg1, deg_in, deg_out, b1)

    agg2 = sc_aggregate(g, ei)
    mu, std, z = _tc_final(agg2, deg_in, eps, W2, b2, W3, b3)

    adj = _tc_decode(z)
    return adj, mu, std


# speedup vs baseline: 1.1754x; 1.0059x over previous
import functools

import jax
import jax.numpy as jnp
from jax import lax
from jax.experimental import pallas as pl
from jax.experimental.pallas import tpu as pltpu
from jax.experimental.pallas import tpu_sc as plsc

N_NODES = 10000
N_EDGES = 160000
F_IN = 128
F_OUT = 32

NCORES = 2
NSUB = 16
NWORK = NCORES * NSUB
CHUNK = 128
EDGES_PER_TILE = 5120
NCHUNK = EDGES_PER_TILE // CHUNK
E_PAD = EDGES_PER_TILE * NWORK
NPAD = 10240
DUMMY = 10016
ROWS_PER_TILE = NPAD // NSUB

def _zero_rows(ref, nrows, ncols):
    zv = jnp.zeros((16,), jnp.float32)

    def body(i, carry):
        for c0 in range(0, ncols, 16):
            ref[i, pl.ds(c0, 16)] = zv
        return carry

    lax.fori_loop(0, nrows, body, 0)


def _sc_degrees_body(edges_hbm, out_hbm, src_idx, dst_idx, ones_v, zbuf,
                     acc_s, acc_d):
    c = lax.axis_index("c")
    s = lax.axis_index("s")
    wid = s * NCORES + c
    ones16 = jnp.ones((16,), jnp.float32)

    def fill_ones(i, carry):
        ones_v[i, :] = ones16
        return carry

    lax.fori_loop(0, CHUNK, fill_ones, 0)
    _zero_rows(zbuf, ROWS_PER_TILE, 16)

    row0 = s * ROWS_PER_TILE
    pltpu.sync_copy(zbuf, acc_s.at[pl.ds(row0, ROWS_PER_TILE)])
    pltpu.sync_copy(zbuf, acc_d.at[pl.ds(row0, ROWS_PER_TILE)])
    plsc.subcore_barrier()

    pltpu.sync_copy(edges_hbm.at[0, wid], src_idx)
    pltpu.sync_copy(edges_hbm.at[1, wid], dst_idx)

    def chunk(j, carry):
        pltpu.sync_copy(ones_v, acc_s.at[src_idx.at[j]], add=True)
        pltpu.sync_copy(ones_v, acc_d.at[dst_idx.at[j]], add=True)
        return carry

    lax.fori_loop(0, NCHUNK, chunk, 0)
    plsc.subcore_barrier()

    pltpu.sync_copy(acc_s.at[pl.ds(row0, ROWS_PER_TILE)],
                    out_hbm.at[c, 0, pl.ds(row0, ROWS_PER_TILE)])
    pltpu.sync_copy(acc_d.at[pl.ds(row0, ROWS_PER_TILE)],
                    out_hbm.at[c, 1, pl.ds(row0, ROWS_PER_TILE)])


NSLOT = 8
NBLK = NCHUNK // NSLOT


def _sc_aggregate_body(h_hbm, edges_hbm, out_hbm, src_idx, dst_idx, rows_v,
                       h_s, acc, sems):
    c = lax.axis_index("c")
    s = lax.axis_index("s")
    wid = s * NCORES + c
    row0 = s * ROWS_PER_TILE

    _zero_rows(rows_v.at[0], CHUNK, F_OUT)
    for m in range(ROWS_PER_TILE // CHUNK):
        pltpu.sync_copy(rows_v.at[0], acc.at[pl.ds(row0 + m * CHUNK, CHUNK)])
    pltpu.sync_copy(h_hbm.at[pl.ds(row0, ROWS_PER_TILE)],
                    h_s.at[pl.ds(row0, ROWS_PER_TILE)])
    pltpu.sync_copy(edges_hbm.at[0, wid], src_idx)
    pltpu.sync_copy(edges_hbm.at[1, wid], dst_idx)
    plsc.subcore_barrier()

    for k in range(NSLOT):
        pltpu.async_copy(h_s.at[src_idx.at[k]], rows_v.at[k], sems.at[k])

    def blk(bb, carry):
        j0 = bb * NSLOT
        for k in range(NSLOT):
            j = j0 + k
            pltpu.make_async_copy(h_s.at[src_idx.at[j]], rows_v.at[k],
                                  sems.at[k]).wait()
            pltpu.sync_copy(rows_v.at[k], acc.at[dst_idx.at[j]], add=True)

            @pl.when(bb < NBLK - 1)
            def _refire():
                pltpu.async_copy(h_s.at[src_idx.at[j + NSLOT]], rows_v.at[k],
                                 sems.at[k])
        return carry

    lax.fori_loop(0, NBLK, blk, 0)
    plsc.subcore_barrier()

    pltpu.sync_copy(acc.at[pl.ds(row0, ROWS_PER_TILE)],
                    out_hbm.at[c, pl.ds(row0, ROWS_PER_TILE)])


@functools.cache
def _sc_kernels():
    mesh = plsc.VectorSubcoreMesh(core_axis_name="c", subcore_axis_name="s",
                                  num_cores=NCORES, num_subcores=NSUB)
    degrees = pl.kernel(
        _sc_degrees_body,
        out_type=jax.ShapeDtypeStruct((NCORES, 2, NPAD, 16), jnp.float32),
        mesh=mesh,
        compiler_params=pltpu.CompilerParams(use_tc_tiling_on_sc=False),
        scratch_types=[
            pltpu.VMEM((NCHUNK, CHUNK), jnp.int32),
            pltpu.VMEM((NCHUNK, CHUNK), jnp.int32),
            pltpu.VMEM((CHUNK, 16), jnp.float32),
            pltpu.VMEM((ROWS_PER_TILE, 16), jnp.float32),
            pltpu.VMEM_SHARED((NPAD, 16), jnp.float32),
            pltpu.VMEM_SHARED((NPAD, 16), jnp.float32),
        ],
    )
    aggregate = pl.kernel(
        _sc_aggregate_body,
        out_type=jax.ShapeDtypeStruct((NCORES, NPAD, F_OUT), jnp.float32),
        mesh=mesh,
        compiler_params=pltpu.CompilerParams(use_tc_tiling_on_sc=False),
        scratch_types=[
            pltpu.VMEM((NCHUNK, CHUNK), jnp.int32),
            pltpu.VMEM((NCHUNK, CHUNK), jnp.int32),
            pltpu.VMEM((NSLOT, CHUNK, F_OUT), jnp.float32),
            pltpu.VMEM_SHARED((NPAD, F_OUT), jnp.float32),
            pltpu.VMEM_SHARED((NPAD, F_OUT), jnp.float32),
            pltpu.SemaphoreType.DMA((NSLOT,)),
        ],
    )
    return degrees, aggregate


def _enc_body(feat_ref, dego_ref, w1_ref, out_ref):
    norm = lax.rsqrt(jnp.maximum(dego_ref[...], 1.0))
    h = feat_ref[...] * norm[:, None]
    out_ref[...] = jnp.dot(h, w1_ref[...], preferred_element_type=jnp.float32)


def _tc_encode(feat, deg_out, w1):
    blk = 1024
    grid = NPAD // blk
    return pl.pallas_call(
        _enc_body,
        grid=(grid,),
        in_specs=[
            pl.BlockSpec((blk, F_IN), lambda i: (i, 0)),
            pl.BlockSpec((blk,), lambda i: (i,)),
            pl.BlockSpec((F_IN, F_OUT), lambda i: (0, 0)),
        ],
        out_specs=pl.BlockSpec((blk, F_OUT), lambda i: (i, 0)),
        out_shape=jax.ShapeDtypeStruct((NPAD, F_OUT), jnp.float32),
    )(feat, deg_out, w1)


def _mid_body(agg_ref, degi_ref, dego_ref, b1_ref, out_ref):
    agg = agg_ref[0] + agg_ref[1]
    ni = lax.rsqrt(jnp.maximum(degi_ref[...], 1.0))
    no = lax.rsqrt(jnp.maximum(dego_ref[...], 1.0))
    h1 = agg * ni[:, None] + b1_ref[...][None, :]
    out_ref[...] = h1 * no[:, None]


def _tc_mid(aggp, deg_in, deg_out, b1):
    blk = 1024
    grid = NPAD // blk
    return pl.pallas_call(
        _mid_body,
        grid=(grid,),
        in_specs=[
            pl.BlockSpec((NCORES, blk, F_OUT), lambda i: (0, i, 0)),
            pl.BlockSpec((blk,), lambda i: (i,)),
            pl.BlockSpec((blk,), lambda i: (i,)),
            pl.BlockSpec((F_OUT,), lambda i: (0,)),
        ],
        out_specs=pl.BlockSpec((blk, F_OUT), lambda i: (i, 0)),
        out_shape=jax.ShapeDtypeStruct((NPAD, F_OUT), jnp.float32),
    )(aggp, deg_in, deg_out, b1)


def _fin_body(agg_ref, degi_ref, eps_ref, w2_ref, b2_ref, w3_ref, b3_ref,
              mu_ref, std_ref, z_ref):
    agg = agg_ref[0] + agg_ref[1]
    ni = lax.rsqrt(jnp.maximum(degi_ref[...], 1.0))
    aggn = agg * ni[:, None]
    mu = jnp.dot(aggn, w2_ref[...], preferred_element_type=jnp.float32) \
        + b2_ref[...][None, :]
    logvar = jnp.dot(aggn, w3_ref[...], preferred_element_type=jnp.float32) \
        + b3_ref[...][None, :]
    std = jnp.exp(logvar)
    mu_ref[...] = mu
    std_ref[...] = std
    z_ref[...] = eps_ref[...] * std + mu


def _tc_final(aggp, deg_in, eps, w2, b2, w3, b3):
    blk = 1024
    grid = pl.cdiv(N_NODES, blk)
    out = jax.ShapeDtypeStruct((N_NODES, F_OUT), jnp.float32)
    return pl.pallas_call(
        _fin_body,
        grid=(grid,),
        in_specs=[
            pl.BlockSpec((NCORES, blk, F_OUT), lambda i: (0, i, 0)),
            pl.BlockSpec((blk,), lambda i: (i,)),
            pl.BlockSpec((blk, F_OUT), lambda i: (i, 0)),
            pl.BlockSpec((F_OUT, F_OUT), lambda i: (0, 0)),
            pl.BlockSpec((F_OUT,), lambda i: (0,)),
            pl.BlockSpec((F_OUT, F_OUT), lambda i: (0, 0)),
            pl.BlockSpec((F_OUT,), lambda i: (0,)),
        ],
        out_specs=[
            pl.BlockSpec((blk, F_OUT), lambda i: (i, 0)),
            pl.BlockSpec((blk, F_OUT), lambda i: (i, 0)),
            pl.BlockSpec((blk, F_OUT), lambda i: (i, 0)),
        ],
        out_shape=[out, out, out],
    )(aggp, deg_in, eps, w2, b2, w3, b3)


def _dec_body(zi_ref, zj_ref, out_ref):
    prod = lax.dot_general(zi_ref[...], zj_ref[...],
                           (((1,), (1,)), ((), ())),
                           preferred_element_type=jnp.float32)
    out_ref[...] = 0.5 * jnp.tanh(0.5 * prod) + 0.5


def _tc_decode(z):
    bi, bj = 2560, 2048
    return pl.pallas_call(
        _dec_body,
        grid=(pl.cdiv(N_NODES, bi), pl.cdiv(N_NODES, bj)),
        in_specs=[
            pl.BlockSpec((bi, F_OUT), lambda i, j: (i, 0)),
            pl.BlockSpec((bj, F_OUT), lambda i, j: (j, 0)),
        ],
        out_specs=pl.BlockSpec((bi, bj), lambda i, j: (i, j)),
        out_shape=jax.ShapeDtypeStruct((N_NODES, N_NODES), jnp.float32),
    )(z, z)


def kernel(feat, edge_index, W1, b1, W2, b2, W3, b3, eps):
    ei = jnp.full((2, E_PAD), DUMMY, jnp.int32)
    ei = ei.at[:, :N_EDGES].set(edge_index)
    ei = ei.reshape(2, NWORK, NCHUNK, CHUNK)

    sc_degrees, sc_aggregate = _sc_kernels()
    degp = sc_degrees(ei)
    deg_out = degp[0, 0, :, 0] + degp[1, 0, :, 0]
    deg_in = degp[0, 1, :, 0] + degp[1, 1, :, 0]

    h = _tc_encode(feat, deg_out, W1)

    agg1 = sc_aggregate(h, ei)
    g = _tc_mid(agg1, deg_in, deg_out, b1)

    agg2 = sc_aggregate(g, ei)
    mu, std, z = _tc_final(agg2, deg_in, eps, W2, b2, W3, b3)

    adj = _tc_decode(z)
    return adj, mu, std
